# Initial kernel scaffold; baseline (speedup 1.0000x reference)
import jax
import jax.numpy as jnp
from jax.experimental import pallas as pl


def kernel(x, pos, edge_index, batch, W_down, b_down, W_up, b_up, W_lin, b_lin):
    raise NotImplementedError("write your pallas kernel here")


# speedup vs baseline: 7.8746x; 7.8746x over previous
import functools
import jax
import jax.numpy as jnp
from jax import lax
from jax.experimental import pallas as pl
from jax.experimental.pallas import tpu as pltpu
from jax.experimental.pallas import tpu_sc as plsc

N = 10000
NPAD = 10240
E = 160000
EB = 128
NBATCH = E // EB
H = 256
HH = 128
RPT = NPAD // 16
RB = 512
GRID = NPAD // RB
KNN = 6
CMAX = 1024

_mesh = plsc.VectorSubcoreMesh(core_axis_name="c", subcore_axis_name="s")


def _nbatches(s):
    return (NBATCH - 1 - s) // 16 + 1


@functools.partial(
    pl.kernel, mesh=_mesh,
    out_type=(jax.ShapeDtypeStruct((NPAD,), jnp.float32),
              jax.ShapeDtypeStruct((NPAD,), jnp.float32)),
    scratch_types=[pltpu.VMEM_SHARED((NPAD,), jnp.float32),
                   pltpu.VMEM((EB,), jnp.float32),
                   pltpu.VMEM((EB,), jnp.int32),
                   pltpu.VMEM((EB,), jnp.int32),
                   pltpu.VMEM((EB,), jnp.int32)],
)
def _stats_sc(src_hbm, dst_hbm, z_hbm, ones_hbm, deg_hbm, kill_hbm,
              acc, onesv, sidx, didx, midx):
    c = lax.axis_index("c")
    s = lax.axis_index("s")
    pltpu.sync_copy(z_hbm, acc.at[pl.ds(s * RPT, RPT)])
    pltpu.sync_copy(ones_hbm, onesv)
    plsc.subcore_barrier()
    nb = _nbatches(s)

    @pl.when(c == 0)
    def _deg():
        def body(i, carry):
            off = (s + 16 * i) * EB
            pltpu.sync_copy(dst_hbm.at[pl.ds(off, EB)], didx)
            pltpu.sync_copy(onesv, acc.at[didx], add=True)
            return carry
        lax.fori_loop(0, nb, body, 0)

    @pl.when(c == 1)
    def _kill():
        def body(i, carry):
            off = (s + 16 * i) * EB
            pltpu.sync_copy(src_hbm.at[pl.ds(off, EB)], sidx)
            pltpu.sync_copy(dst_hbm.at[pl.ds(off, EB)], didx)
            for j in range(EB // 16):
                sv = sidx[pl.ds(16 * j, 16)]
                dv = didx[pl.ds(16 * j, 16)]
                midx[pl.ds(16 * j, 16)] = jnp.maximum(sv, dv)
            pltpu.sync_copy(onesv, acc.at[midx], add=True)
            return carry
        lax.fori_loop(0, nb, body, 0)

    plsc.subcore_barrier()

    @pl.when(c == 0)
    def _out0():
        pltpu.sync_copy(acc.at[pl.ds(s * RPT, RPT)],
                        deg_hbm.at[pl.ds(s * RPT, RPT)])

    @pl.when(c == 1)
    def _out1():
        pltpu.sync_copy(acc.at[pl.ds(s * RPT, RPT)],
                        kill_hbm.at[pl.ds(s * RPT, RPT)])


@functools.partial(
    pl.kernel, mesh=_mesh,
    out_type=(jax.ShapeDtypeStruct((NPAD, HH), jnp.float32),
              jax.ShapeDtypeStruct((NPAD, HH), jnp.float32)),
    scratch_types=[pltpu.VMEM_SHARED((NPAD, HH), jnp.float32),
                   pltpu.VMEM((EB, HH), jnp.float32),
                   pltpu.VMEM((EB,), jnp.int32),
                   pltpu.VMEM((EB,), jnp.int32),
                   pltpu.SemaphoreType.DMA],
)
def _segsum_sc(hp0_hbm, hp1_hbm, src_hbm, dst_hbm, z2_hbm, s0_hbm, s1_hbm,
               acc, rows, sidx, didx, sem):
    c = lax.axis_index("c")
    s = lax.axis_index("s")
    pltpu.sync_copy(z2_hbm, acc.at[pl.ds(s * RPT, RPT)])
    plsc.subcore_barrier()
    nb = _nbatches(s)

    def body(i, carry):
        off = (s + 16 * i) * EB
        pltpu.sync_copy(src_hbm.at[pl.ds(off, EB)], sidx)
        pltpu.sync_copy(dst_hbm.at[pl.ds(off, EB)], didx)

        @pl.when(c == 0)
        def _g0():
            pltpu.async_copy(hp0_hbm.at[sidx], rows, sem).wait()

        @pl.when(c == 1)
        def _g1():
            pltpu.async_copy(hp1_hbm.at[sidx], rows, sem).wait()

        pltpu.sync_copy(rows, acc.at[didx], add=True)
        return carry

    lax.fori_loop(0, nb, body, 0)
    plsc.subcore_barrier()

    @pl.when(c == 0)
    def _out0():
        pltpu.sync_copy(acc.at[pl.ds(s * RPT, RPT)],
                        s0_hbm.at[pl.ds(s * RPT, RPT)])

    @pl.when(c == 1)
    def _out1():
        pltpu.sync_copy(acc.at[pl.ds(s * RPT, RPT)],
                        s1_hbm.at[pl.ds(s * RPT, RPT)])


@functools.partial(
    pl.kernel, mesh=_mesh,
    out_type=jax.ShapeDtypeStruct((CMAX,), jnp.float32),
    scratch_types=[pltpu.VMEM_SHARED((2 * CMAX,), jnp.float32),
                   pltpu.VMEM((RPT,), jnp.float32),
                   pltpu.VMEM((RPT,), jnp.float32),
                   pltpu.VMEM((EB,), jnp.int32),
                   pltpu.VMEM((EB,), jnp.float32)],
)
def _scan_sc(cand_hbm, rank_hbm, z_hbm, clist_hbm, acc2, cb, rb, idx, val):
    c = lax.axis_index("c")
    s = lax.axis_index("s")

    @pl.when(c == 0)
    def _run():
        pltpu.sync_copy(z_hbm.at[pl.ds(0, EB)], acc2.at[pl.ds(s * EB, EB)])
        pltpu.sync_copy(cand_hbm.at[pl.ds(s * RPT, RPT)], cb)
        pltpu.sync_copy(rank_hbm.at[pl.ds(s * RPT, RPT)], rb)
        plsc.subcore_barrier()

        def body(i, carry):
            for j in range(EB // 16):
                o = i * EB + 16 * j
                v = cb[pl.ds(o, 16)]
                rk = rb[pl.ds(o, 16)].astype(jnp.int32)
                nidx = lax.iota(jnp.int32, 16) + (s * RPT + o)
                trash = CMAX + jnp.bitwise_and(nidx, CMAX - 1)
                idx[pl.ds(16 * j, 16)] = jnp.where(v > 0.5, rk, trash)
                val[pl.ds(16 * j, 16)] = nidx.astype(jnp.float32)
            pltpu.sync_copy(val, acc2.at[idx], add=True)
            return carry

        lax.fori_loop(0, RPT // EB, body, 0)
        plsc.subcore_barrier()

        @pl.when(s < CMAX // EB)
        def _out():
            pltpu.sync_copy(acc2.at[pl.ds(s * EB, EB)],
                            clist_hbm.at[pl.ds(s * EB, EB)])


@functools.partial(
    pl.kernel, mesh=_mesh,
    out_type=(jax.ShapeDtypeStruct((CMAX, H), jnp.float32),
              jax.ShapeDtypeStruct((CMAX, 128), jnp.float32)),
    scratch_types=[pltpu.VMEM((EB,), jnp.int32),
                   pltpu.VMEM((EB,), jnp.float32),
                   pltpu.VMEM((EB, H), jnp.float32),
                   pltpu.VMEM((EB, 128), jnp.float32),
                   pltpu.SemaphoreType.DMA],
)
def _gather_sc(clist_hbm, x1_hbm, pos_hbm, x1c_hbm, posc_hbm,
               idxb, clf, rx, rp, sem):
    c = lax.axis_index("c")
    s = lax.axis_index("s")

    @pl.when(jnp.logical_and(c == 0, s < CMAX // EB))
    def _run():
        pltpu.sync_copy(clist_hbm.at[pl.ds(s * EB, EB)], clf)
        for j in range(EB // 16):
            idxb[pl.ds(16 * j, 16)] = clf[pl.ds(16 * j, 16)].astype(
                jnp.int32)
        pltpu.async_copy(x1_hbm.at[idxb], rx, sem).wait()
        pltpu.sync_copy(rx, x1c_hbm.at[pl.ds(s * EB, EB)])
        pltpu.async_copy(pos_hbm.at[idxb], rp, sem).wait()
        pltpu.sync_copy(rp, posc_hbm.at[pl.ds(s * EB, EB)])


def _conv_prep_body(x_ref, w_ref, b_ref, deg_ref, kill_ref,
                    hp0_ref, hp1_ref, p1_ref, dinv_ref, cand_ref,
                    rank_ref, mc_ref, carry_ref):
    i = pl.program_id(0)
    h = jnp.dot(x_ref[...], w_ref[...], preferred_element_type=jnp.float32)
    dinv = lax.rsqrt(deg_ref[...] + 2.0)
    hp = dinv * h
    hp0_ref[...] = hp[:, :HH]
    hp1_ref[...] = hp[:, HH:]
    p1_ref[...] = 2.0 * dinv * dinv * h + b_ref[...]
    dinv_ref[...] = dinv

    @pl.when(i == 0)
    def _init():
        carry_ref[0, 0] = 0.0

    ridx = lax.broadcasted_iota(jnp.int32, (RB, 1), 0) + i * RB
    keep = jnp.where(
        jnp.logical_and(kill_ref[...] == 0.0, ridx < N), 1.0, 0.0)
    rr = lax.broadcasted_iota(jnp.int32, (RB, RB), 0)
    cc = lax.broadcasted_iota(jnp.int32, (RB, RB), 1)
    tri = jnp.where(cc <= rr, 1.0, 0.0)
    p = jnp.dot(tri, keep, preferred_element_type=jnp.float32)
    carry = carry_ref[0, 0]
    rank = p - keep + carry
    cand_ref[...] = keep * jnp.where(rank < 1000.0, 1.0, 0.0)
    rank_ref[...] = rank
    new_carry = carry + jnp.sum(keep)
    carry_ref[0, 0] = new_carry

    @pl.when(i == GRID - 1)
    def _emit_m():
        mc_ref[...] = jnp.zeros((1, 16), jnp.float32) + jnp.minimum(
            new_carry, 1000.0)


def _conv_prep(x, w, b2, deg2, kill2):
    return pl.pallas_call(
        _conv_prep_body,
        grid=(GRID,),
        in_specs=[pl.BlockSpec((RB, H), lambda i: (i, 0)),
                  pl.BlockSpec((H, H), lambda i: (0, 0)),
                  pl.BlockSpec((1, H), lambda i: (0, 0)),
                  pl.BlockSpec((RB, 1), lambda i: (i, 0)),
                  pl.BlockSpec((RB, 1), lambda i: (i, 0))],
        out_specs=[pl.BlockSpec((RB, HH), lambda i: (i, 0)),
                   pl.BlockSpec((RB, HH), lambda i: (i, 0)),
                   pl.BlockSpec((RB, H), lambda i: (i, 0)),
                   pl.BlockSpec((RB, 1), lambda i: (i, 0)),
                   pl.BlockSpec((RB, 1), lambda i: (i, 0)),
                   pl.BlockSpec((RB, 1), lambda i: (i, 0)),
                   pl.BlockSpec((1, 16), lambda i: (0, 0))],
        out_shape=[jax.ShapeDtypeStruct((NPAD, HH), jnp.float32),
                   jax.ShapeDtypeStruct((NPAD, HH), jnp.float32),
                   jax.ShapeDtypeStruct((NPAD, H), jnp.float32),
                   jax.ShapeDtypeStruct((NPAD, 1), jnp.float32),
                   jax.ShapeDtypeStruct((NPAD, 1), jnp.float32),
                   jax.ShapeDtypeStruct((NPAD, 1), jnp.float32),
                   jax.ShapeDtypeStruct((1, 16), jnp.float32)],
        scratch_shapes=[pltpu.SMEM((1, 1), jnp.float32)],
    )(x, w, b2, deg2, kill2)


def _finish_body(s0_ref, s1_ref, p_ref, dinv_ref, o_ref):
    sall = jnp.concatenate([s0_ref[...], s1_ref[...]], axis=1)
    o_ref[...] = jnp.maximum(dinv_ref[...] * sall + p_ref[...], 0.0)


def _conv_finish(s0, s1, p, dinv2):
    return pl.pallas_call(
        _finish_body,
        grid=(GRID,),
        in_specs=[pl.BlockSpec((RB, HH), lambda i: (i, 0)),
                  pl.BlockSpec((RB, HH), lambda i: (i, 0)),
                  pl.BlockSpec((RB, H), lambda i: (i, 0)),
                  pl.BlockSpec((RB, 1), lambda i: (i, 0))],
        out_specs=pl.BlockSpec((RB, H), lambda i: (i, 0)),
        out_shape=jax.ShapeDtypeStruct((NPAD, H), jnp.float32),
    )(s0, s1, p, dinv2)


def _interp_body(posq_ref, posc_ref, m_ref, x1c_ref, wup_ref, bup_ref,
                 dinv_ref, hp0_ref, hp1_ref, p2_ref):
    pq = posq_ref[...]
    pc = posc_ref[...]
    dmat = jnp.zeros((RB, CMAX), jnp.float32)
    for k in range(3):
        diff = pq[:, k:k + 1] - pc[:, k][None, :]
        dmat = dmat + diff * diff
    m = m_ref[0, 0]
    col = lax.broadcasted_iota(jnp.int32, (RB, CMAX), 1).astype(jnp.float32)
    dmat = jnp.where(col < m, dmat, jnp.inf)
    wmat = jnp.zeros((RB, CMAX), jnp.float32)
    den = jnp.zeros((RB, 1), jnp.float32)
    for _ in range(KNN):
        mn = jnp.min(dmat, axis=1, keepdims=True)
        w = 1.0 / jnp.maximum(mn, 1e-16)
        cstar = jnp.min(jnp.where(dmat <= mn, col, 2.0 * CMAX),
                        axis=1, keepdims=True)
        sel = col == cstar
        wmat = wmat + jnp.where(sel, w, 0.0)
        den = den + w
        dmat = jnp.where(sel, jnp.inf, dmat)
    xu = jnp.dot(wmat, x1c_ref[...],
                 preferred_element_type=jnp.float32) / den
    h2 = jnp.dot(xu, wup_ref[...], preferred_element_type=jnp.float32)
    dinv = dinv_ref[...]
    hp = dinv * h2
    hp0_ref[...] = hp[:, :HH]
    hp1_ref[...] = hp[:, HH:]
    p2_ref[...] = 2.0 * dinv * dinv * h2 + bup_ref[...]


def _interp(pos16, posc, m16, x1c, wup, bup2, dinv2):
    return pl.pallas_call(
        _interp_body,
        grid=(GRID,),
        in_specs=[pl.BlockSpec((RB, 16), lambda i: (i, 0)),
                  pl.BlockSpec((CMAX, 128), lambda i: (0, 0)),
                  pl.BlockSpec((1, 16), lambda i: (0, 0)),
                  pl.BlockSpec((CMAX, H), lambda i: (0, 0)),
                  pl.BlockSpec((H, H), lambda i: (0, 0)),
                  pl.BlockSpec((1, H), lambda i: (0, 0)),
                  pl.BlockSpec((RB, 1), lambda i: (i, 0))],
        out_specs=[pl.BlockSpec((RB, HH), lambda i: (i, 0)),
                   pl.BlockSpec((RB, HH), lambda i: (i, 0)),
                   pl.BlockSpec((RB, H), lambda i: (i, 0))],
        out_shape=[jax.ShapeDtypeStruct((NPAD, HH), jnp.float32),
                   jax.ShapeDtypeStruct((NPAD, HH), jnp.float32),
                   jax.ShapeDtypeStruct((NPAD, H), jnp.float32)],
    )(pos16, posc, m16, x1c, wup, bup2, dinv2)


def _final_body(s0_ref, s1_ref, p_ref, dinv_ref, wl_ref, bl_ref, o_ref):
    sall = jnp.concatenate([s0_ref[...], s1_ref[...]], axis=1)
    xu2 = jnp.maximum(dinv_ref[...] * sall + p_ref[...], 0.0)
    o_ref[...] = jnp.dot(xu2, wl_ref[...],
                         preferred_element_type=jnp.float32) + bl_ref[...]


def _final(s0, s1, p, dinv2, wl, bl2):
    return pl.pallas_call(
        _final_body,
        grid=(GRID,),
        in_specs=[pl.BlockSpec((RB, HH), lambda i: (i, 0)),
                  pl.BlockSpec((RB, HH), lambda i: (i, 0)),
                  pl.BlockSpec((RB, H), lambda i: (i, 0)),
                  pl.BlockSpec((RB, 1), lambda i: (i, 0)),
                  pl.BlockSpec((H, H), lambda i: (0, 0)),
                  pl.BlockSpec((1, H), lambda i: (0, 0))],
        out_specs=pl.BlockSpec((RB, H), lambda i: (i, 0)),
        out_shape=jax.ShapeDtypeStruct((NPAD, H), jnp.float32),
    )(s0, s1, p, dinv2, wl, bl2)


@jax.jit
def kernel(x, pos, edge_index, batch, W_down, b_down, W_up, b_up,
           W_lin, b_lin):
    src = edge_index[0].astype(jnp.int32)
    dst = edge_index[1].astype(jnp.int32)
    x_pad = jnp.pad(x, ((0, NPAD - N), (0, 0)))
    pos16 = jnp.pad(pos, ((0, NPAD - N), (0, 13)))
    pos128 = jnp.pad(pos, ((0, NPAD - N), (0, 125)))
    z1 = jnp.zeros((RPT,), jnp.float32)
    z2 = jnp.zeros((RPT, HH), jnp.float32)
    ones = jnp.ones((EB,), jnp.float32)

    deg, kill = _stats_sc(src, dst, z1, ones)
    hp0, hp1, p1, dinv2, cand2, rank2, mcount = _conv_prep(
        x_pad, W_down, b_down[None, :], deg[:, None], kill[:, None])
    s0, s1 = _segsum_sc(hp0, hp1, src, dst, z2)
    clist = _scan_sc(cand2[:, 0], rank2[:, 0], z1)
    x1 = _conv_finish(s0, s1, p1, dinv2)
    x1c, posc = _gather_sc(clist, x1, pos128)
    h2p0, h2p1, p2 = _interp(pos16, posc, mcount, x1c,
                             W_up, b_up[None, :], dinv2)
    s20, s21 = _segsum_sc(h2p0, h2p1, src, dst, z2)
    out = _final(s20, s21, p2, dinv2, W_lin, b_lin[None, :])
    return out[:N]


# speedup vs baseline: 11.0373x; 1.4016x over previous
import functools
import jax
import jax.numpy as jnp
from jax import lax
from jax.experimental import pallas as pl
from jax.experimental.pallas import tpu as pltpu
from jax.experimental.pallas import tpu_sc as plsc

N = 10000
NPAD = 10240
E = 160000
EB = 128
NBATCH = E // EB
BPT = 79
EPAD = 161792
H = 256
HH = 128
RPT = NPAD // 16
RB = 512
GRID = NPAD // RB
KNN = 6
CMAX = 1024

_mesh = plsc.VectorSubcoreMesh(core_axis_name="c", subcore_axis_name="s")


def _nbatches(s):
    return (NBATCH - 1 - s) // 16 + 1


@functools.partial(
    pl.kernel, mesh=_mesh,
    out_type=(jax.ShapeDtypeStruct((NPAD,), jnp.float32),
              jax.ShapeDtypeStruct((NPAD,), jnp.float32)),
    scratch_types=[pltpu.VMEM_SHARED((NPAD,), jnp.float32),
                   pltpu.VMEM((EB,), jnp.float32),
                   pltpu.VMEM((EB,), jnp.int32),
                   pltpu.VMEM((EB,), jnp.int32),
                   pltpu.VMEM((EB,), jnp.int32)],
)
def _stats_sc(src_hbm, dst_hbm, z_hbm, ones_hbm, deg_hbm, kill_hbm,
              acc, onesv, sidx, didx, midx):
    c = lax.axis_index("c")
    s = lax.axis_index("s")
    pltpu.sync_copy(z_hbm, acc.at[pl.ds(s * RPT, RPT)])
    pltpu.sync_copy(ones_hbm, onesv)
    plsc.subcore_barrier()
    nb = _nbatches(s)

    @pl.when(c == 0)
    def _deg():
        def body(i, carry):
            off = (s + 16 * i) * EB
            pltpu.sync_copy(dst_hbm.at[pl.ds(off, EB)], didx)
            pltpu.sync_copy(onesv, acc.at[didx], add=True)
            return carry
        lax.fori_loop(0, nb, body, 0)

    @pl.when(c == 1)
    def _kill():
        def body(i, carry):
            off = (s + 16 * i) * EB
            pltpu.sync_copy(src_hbm.at[pl.ds(off, EB)], sidx)
            pltpu.sync_copy(dst_hbm.at[pl.ds(off, EB)], didx)
            for j in range(EB // 16):
                sv = sidx[pl.ds(16 * j, 16)]
                dv = didx[pl.ds(16 * j, 16)]
                midx[pl.ds(16 * j, 16)] = jnp.maximum(sv, dv)
            pltpu.sync_copy(onesv, acc.at[midx], add=True)
            return carry
        lax.fori_loop(0, nb, body, 0)

    plsc.subcore_barrier()

    @pl.when(c == 0)
    def _out0():
        pltpu.sync_copy(acc.at[pl.ds(s * RPT, RPT)],
                        deg_hbm.at[pl.ds(s * RPT, RPT)])

    @pl.when(c == 1)
    def _out1():
        pltpu.sync_copy(acc.at[pl.ds(s * RPT, RPT)],
                        kill_hbm.at[pl.ds(s * RPT, RPT)])


@functools.partial(
    pl.kernel, mesh=_mesh,
    out_type=(jax.ShapeDtypeStruct((NPAD, HH), jnp.float32),
              jax.ShapeDtypeStruct((NPAD, HH), jnp.float32)),
    scratch_types=[pltpu.VMEM_SHARED((NPAD, HH), jnp.float32),
                   pltpu.VMEM((EB, HH), jnp.float32),
                   pltpu.VMEM((EB, HH), jnp.float32),
                   pltpu.VMEM((BPT * EB,), jnp.int32),
                   pltpu.VMEM((EB,), jnp.int32),
                   pltpu.VMEM((EB,), jnp.int32),
                   pltpu.SemaphoreType.DMA,
                   pltpu.SemaphoreType.DMA],
)
def _segsum_sc(hp0_hbm, hp1_hbm, src_hbm, dst_hbm, z2_hbm, s0_hbm, s1_hbm,
               acc, rows_a, rows_b, sall, didx_a, didx_b, sem_a, sem_b):
    c = lax.axis_index("c")
    s = lax.axis_index("s")
    pltpu.sync_copy(z2_hbm, acc.at[pl.ds(s * RPT, RPT)])
    start = 78 * s + jnp.minimum(s, 2)
    nb = 78 + jnp.where(s < 2, 1, 0)
    pltpu.sync_copy(src_hbm.at[pl.ds(start * EB, BPT * EB)], sall)
    plsc.subcore_barrier()

    def gather(bi, rows, sem):
        idx = sall.at[pl.ds(bi * EB, EB)]

        @pl.when(c == 0)
        def _g0():
            pltpu.async_copy(hp0_hbm.at[idx], rows, sem)

        @pl.when(c == 1)
        def _g1():
            pltpu.async_copy(hp1_hbm.at[idx], rows, sem)

    def drain(rows, sem):
        pltpu.make_async_copy(hp0_hbm.at[pl.ds(0, EB)], rows, sem).wait()

    def scatter(bi, rows, didx):
        pltpu.sync_copy(dst_hbm.at[pl.ds((start + bi) * EB, EB)], didx)
        pltpu.sync_copy(rows, acc.at[didx], add=True)

    gather(0, rows_a, sem_a)

    def body(i, carry):
        gather(2 * i + 1, rows_b, sem_b)
        drain(rows_a, sem_a)
        scatter(2 * i, rows_a, didx_a)

        @pl.when(2 * i + 2 < nb)
        def _pf():
            gather(2 * i + 2, rows_a, sem_a)

        drain(rows_b, sem_b)
        scatter(2 * i + 1, rows_b, didx_b)
        return carry

    lax.fori_loop(0, 39, body, 0)

    @pl.when(nb > 78)
    def _tail():
        drain(rows_a, sem_a)
        scatter(78, rows_a, didx_a)

    plsc.subcore_barrier()

    @pl.when(c == 0)
    def _out0():
        pltpu.sync_copy(acc.at[pl.ds(s * RPT, RPT)],
                        s0_hbm.at[pl.ds(s * RPT, RPT)])

    @pl.when(c == 1)
    def _out1():
        pltpu.sync_copy(acc.at[pl.ds(s * RPT, RPT)],
                        s1_hbm.at[pl.ds(s * RPT, RPT)])


@functools.partial(
    pl.kernel, mesh=_mesh,
    out_type=jax.ShapeDtypeStruct((CMAX,), jnp.float32),
    scratch_types=[pltpu.VMEM_SHARED((2 * CMAX,), jnp.float32),
                   pltpu.VMEM((RPT,), jnp.float32),
                   pltpu.VMEM((RPT,), jnp.float32),
                   pltpu.VMEM((EB,), jnp.int32),
                   pltpu.VMEM((EB,), jnp.float32)],
)
def _scan_sc(cand_hbm, rank_hbm, z_hbm, clist_hbm, acc2, cb, rb, idx, val):
    c = lax.axis_index("c")
    s = lax.axis_index("s")

    @pl.when(c == 0)
    def _run():
        pltpu.sync_copy(z_hbm.at[pl.ds(0, EB)], acc2.at[pl.ds(s * EB, EB)])
        pltpu.sync_copy(cand_hbm.at[pl.ds(s * RPT, RPT)], cb)
        pltpu.sync_copy(rank_hbm.at[pl.ds(s * RPT, RPT)], rb)
        plsc.subcore_barrier()

        def body(i, carry):
            for j in range(EB // 16):
                o = i * EB + 16 * j
                v = cb[pl.ds(o, 16)]
                rk = rb[pl.ds(o, 16)].astype(jnp.int32)
                nidx = lax.iota(jnp.int32, 16) + (s * RPT + o)
                trash = CMAX + jnp.bitwise_and(nidx, CMAX - 1)
                idx[pl.ds(16 * j, 16)] = jnp.where(v > 0.5, rk, trash)
                val[pl.ds(16 * j, 16)] = nidx.astype(jnp.float32)
            pltpu.sync_copy(val, acc2.at[idx], add=True)
            return carry

        lax.fori_loop(0, RPT // EB, body, 0)
        plsc.subcore_barrier()

        @pl.when(s < CMAX // EB)
        def _out():
            pltpu.sync_copy(acc2.at[pl.ds(s * EB, EB)],
                            clist_hbm.at[pl.ds(s * EB, EB)])


@functools.partial(
    pl.kernel, mesh=_mesh,
    out_type=(jax.ShapeDtypeStruct((CMAX, H), jnp.float32),
              jax.ShapeDtypeStruct((CMAX, 128), jnp.float32)),
    scratch_types=[pltpu.VMEM((EB,), jnp.int32),
                   pltpu.VMEM((EB,), jnp.float32),
                   pltpu.VMEM((EB, H), jnp.float32),
                   pltpu.VMEM((EB, 128), jnp.float32),
                   pltpu.SemaphoreType.DMA],
)
def _gather_sc(clist_hbm, x1_hbm, pos_hbm, x1c_hbm, posc_hbm,
               idxb, clf, rx, rp, sem):
    c = lax.axis_index("c")
    s = lax.axis_index("s")

    @pl.when(jnp.logical_and(c == 0, s < CMAX // EB))
    def _run():
        pltpu.sync_copy(clist_hbm.at[pl.ds(s * EB, EB)], clf)
        for j in range(EB // 16):
            idxb[pl.ds(16 * j, 16)] = clf[pl.ds(16 * j, 16)].astype(
                jnp.int32)
        pltpu.async_copy(x1_hbm.at[idxb], rx, sem).wait()
        pltpu.sync_copy(rx, x1c_hbm.at[pl.ds(s * EB, EB)])
        pltpu.async_copy(pos_hbm.at[idxb], rp, sem).wait()
        pltpu.sync_copy(rp, posc_hbm.at[pl.ds(s * EB, EB)])


def _conv_prep_body(x_ref, w_ref, b_ref, deg_ref, kill_ref,
                    hp0_ref, hp1_ref, p1_ref, dinv_ref, cand_ref,
                    rank_ref, mc_ref, carry_ref):
    i = pl.program_id(0)
    h = jnp.dot(x_ref[...], w_ref[...], preferred_element_type=jnp.float32)
    dinv = lax.rsqrt(deg_ref[...] + 2.0)
    hp = dinv * h
    hp0_ref[...] = hp[:, :HH]
    hp1_ref[...] = hp[:, HH:]
    p1_ref[...] = 2.0 * dinv * dinv * h + b_ref[...]
    dinv_ref[...] = dinv

    @pl.when(i == 0)
    def _init():
        carry_ref[0, 0] = 0.0

    ridx = lax.broadcasted_iota(jnp.int32, (RB, 1), 0) + i * RB
    keep = jnp.where(
        jnp.logical_and(kill_ref[...] == 0.0, ridx < N), 1.0, 0.0)
    rr = lax.broadcasted_iota(jnp.int32, (RB, RB), 0)
    cc = lax.broadcasted_iota(jnp.int32, (RB, RB), 1)
    tri = jnp.where(cc <= rr, 1.0, 0.0)
    p = jnp.dot(tri, keep, preferred_element_type=jnp.float32)
    carry = carry_ref[0, 0]
    rank = p - keep + carry
    cand_ref[...] = keep * jnp.where(rank < 1000.0, 1.0, 0.0)
    rank_ref[...] = rank
    new_carry = carry + jnp.sum(keep)
    carry_ref[0, 0] = new_carry

    @pl.when(i == GRID - 1)
    def _emit_m():
        mc_ref[...] = jnp.zeros((1, 16), jnp.float32) + jnp.minimum(
            new_carry, 1000.0)


def _conv_prep(x, w, b2, deg2, kill2):
    return pl.pallas_call(
        _conv_prep_body,
        grid=(GRID,),
        in_specs=[pl.BlockSpec((RB, H), lambda i: (i, 0)),
                  pl.BlockSpec((H, H), lambda i: (0, 0)),
                  pl.BlockSpec((1, H), lambda i: (0, 0)),
                  pl.BlockSpec((RB, 1), lambda i: (i, 0)),
                  pl.BlockSpec((RB, 1), lambda i: (i, 0))],
        out_specs=[pl.BlockSpec((RB, HH), lambda i: (i, 0)),
                   pl.BlockSpec((RB, HH), lambda i: (i, 0)),
                   pl.BlockSpec((RB, H), lambda i: (i, 0)),
                   pl.BlockSpec((RB, 1), lambda i: (i, 0)),
                   pl.BlockSpec((RB, 1), lambda i: (i, 0)),
                   pl.BlockSpec((RB, 1), lambda i: (i, 0)),
                   pl.BlockSpec((1, 16), lambda i: (0, 0))],
        out_shape=[jax.ShapeDtypeStruct((NPAD, HH), jnp.float32),
                   jax.ShapeDtypeStruct((NPAD, HH), jnp.float32),
                   jax.ShapeDtypeStruct((NPAD, H), jnp.float32),
                   jax.ShapeDtypeStruct((NPAD, 1), jnp.float32),
                   jax.ShapeDtypeStruct((NPAD, 1), jnp.float32),
                   jax.ShapeDtypeStruct((NPAD, 1), jnp.float32),
                   jax.ShapeDtypeStruct((1, 16), jnp.float32)],
        scratch_shapes=[pltpu.SMEM((1, 1), jnp.float32)],
    )(x, w, b2, deg2, kill2)


def _finish_body(s0_ref, s1_ref, p_ref, dinv_ref, o_ref):
    sall = jnp.concatenate([s0_ref[...], s1_ref[...]], axis=1)
    o_ref[...] = jnp.maximum(dinv_ref[...] * sall + p_ref[...], 0.0)


def _conv_finish(s0, s1, p, dinv2):
    return pl.pallas_call(
        _finish_body,
        grid=(GRID,),
        in_specs=[pl.BlockSpec((RB, HH), lambda i: (i, 0)),
                  pl.BlockSpec((RB, HH), lambda i: (i, 0)),
                  pl.BlockSpec((RB, H), lambda i: (i, 0)),
                  pl.BlockSpec((RB, 1), lambda i: (i, 0))],
        out_specs=pl.BlockSpec((RB, H), lambda i: (i, 0)),
        out_shape=jax.ShapeDtypeStruct((NPAD, H), jnp.float32),
    )(s0, s1, p, dinv2)


def _interp_body(posq_ref, posc_ref, m_ref, x1c_ref, wup_ref, bup_ref,
                 dinv_ref, hp0_ref, hp1_ref, p2_ref):
    pq = posq_ref[...]
    pc = posc_ref[...]
    dmat = jnp.zeros((RB, CMAX), jnp.float32)
    for k in range(3):
        diff = pq[:, k:k + 1] - pc[:, k][None, :]
        dmat = dmat + diff * diff
    m = m_ref[0, 0]
    col = lax.broadcasted_iota(jnp.int32, (RB, CMAX), 1).astype(jnp.float32)
    dmat = jnp.where(col < m, dmat, jnp.inf)
    wmat = jnp.zeros((RB, CMAX), jnp.float32)
    den = jnp.zeros((RB, 1), jnp.float32)
    for _ in range(KNN):
        mn = jnp.min(dmat, axis=1, keepdims=True)
        w = 1.0 / jnp.maximum(mn, 1e-16)
        cstar = jnp.min(jnp.where(dmat <= mn, col, 2.0 * CMAX),
                        axis=1, keepdims=True)
        sel = col == cstar
        wmat = wmat + jnp.where(sel, w, 0.0)
        den = den + w
        dmat = jnp.where(sel, jnp.inf, dmat)
    xu = jnp.dot(wmat, x1c_ref[...],
                 preferred_element_type=jnp.float32) / den
    h2 = jnp.dot(xu, wup_ref[...], preferred_element_type=jnp.float32)
    dinv = dinv_ref[...]
    hp = dinv * h2
    hp0_ref[...] = hp[:, :HH]
    hp1_ref[...] = hp[:, HH:]
    p2_ref[...] = 2.0 * dinv * dinv * h2 + bup_ref[...]


def _interp(pos16, posc, m16, x1c, wup, bup2, dinv2):
    return pl.pallas_call(
        _interp_body,
        grid=(GRID,),
        in_specs=[pl.BlockSpec((RB, 16), lambda i: (i, 0)),
                  pl.BlockSpec((CMAX, 128), lambda i: (0, 0)),
                  pl.BlockSpec((1, 16), lambda i: (0, 0)),
                  pl.BlockSpec((CMAX, H), lambda i: (0, 0)),
                  pl.BlockSpec((H, H), lambda i: (0, 0)),
                  pl.BlockSpec((1, H), lambda i: (0, 0)),
                  pl.BlockSpec((RB, 1), lambda i: (i, 0))],
        out_specs=[pl.BlockSpec((RB, HH), lambda i: (i, 0)),
                   pl.BlockSpec((RB, HH), lambda i: (i, 0)),
                   pl.BlockSpec((RB, H), lambda i: (i, 0))],
        out_shape=[jax.ShapeDtypeStruct((NPAD, HH), jnp.float32),
                   jax.ShapeDtypeStruct((NPAD, HH), jnp.float32),
                   jax.ShapeDtypeStruct((NPAD, H), jnp.float32)],
    )(pos16, posc, m16, x1c, wup, bup2, dinv2)


def _final_body(s0_ref, s1_ref, p_ref, dinv_ref, wl_ref, bl_ref, o_ref):
    sall = jnp.concatenate([s0_ref[...], s1_ref[...]], axis=1)
    xu2 = jnp.maximum(dinv_ref[...] * sall + p_ref[...], 0.0)
    o_ref[...] = jnp.dot(xu2, wl_ref[...],
                         preferred_element_type=jnp.float32) + bl_ref[...]


def _final(s0, s1, p, dinv2, wl, bl2):
    return pl.pallas_call(
        _final_body,
        grid=(GRID,),
        in_specs=[pl.BlockSpec((RB, HH), lambda i: (i, 0)),
                  pl.BlockSpec((RB, HH), lambda i: (i, 0)),
                  pl.BlockSpec((RB, H), lambda i: (i, 0)),
                  pl.BlockSpec((RB, 1), lambda i: (i, 0)),
                  pl.BlockSpec((H, H), lambda i: (0, 0)),
                  pl.BlockSpec((1, H), lambda i: (0, 0))],
        out_specs=pl.BlockSpec((RB, H), lambda i: (i, 0)),
        out_shape=jax.ShapeDtypeStruct((NPAD, H), jnp.float32),
    )(s0, s1, p, dinv2, wl, bl2)


@jax.jit
def kernel(x, pos, edge_index, batch, W_down, b_down, W_up, b_up,
           W_lin, b_lin):
    src = jnp.pad(edge_index[0].astype(jnp.int32), (0, EPAD - E))
    dst = jnp.pad(edge_index[1].astype(jnp.int32), (0, EPAD - E))
    x_pad = jnp.pad(x, ((0, NPAD - N), (0, 0)))
    pos16 = jnp.pad(pos, ((0, NPAD - N), (0, 13)))
    pos128 = jnp.pad(pos, ((0, NPAD - N), (0, 125)))
    z1 = jnp.zeros((RPT,), jnp.float32)
    z2 = jnp.zeros((RPT, HH), jnp.float32)
    ones = jnp.ones((EB,), jnp.float32)

    deg, kill = _stats_sc(src, dst, z1, ones)
    hp0, hp1, p1, dinv2, cand2, rank2, mcount = _conv_prep(
        x_pad, W_down, b_down[None, :], deg[:, None], kill[:, None])
    s0, s1 = _segsum_sc(hp0, hp1, src, dst, z2)
    clist = _scan_sc(cand2[:, 0], rank2[:, 0], z1)
    x1 = _conv_finish(s0, s1, p1, dinv2)
    x1c, posc = _gather_sc(clist, x1, pos128)
    h2p0, h2p1, p2 = _interp(pos16, posc, mcount, x1c,
                             W_up, b_up[None, :], dinv2)
    s20, s21 = _segsum_sc(h2p0, h2p1, src, dst, z2)
    out = _final(s20, s21, p2, dinv2, W_lin, b_lin[None, :])
    return out[:N]


# speedup vs baseline: 12.9645x; 1.1746x over previous
import functools
import jax
import jax.numpy as jnp
from jax import lax
from jax.experimental import pallas as pl
from jax.experimental.pallas import tpu as pltpu
from jax.experimental.pallas import tpu_sc as plsc

N = 10000
NPAD = 10240
E = 160000
EB = 128
NBATCH = E // EB
BPT = 79
EPAD = 161792
H = 256
HH = 128
RPT = NPAD // 16
RB = 512
GRID = NPAD // RB
KNN = 6
CMAX = 1024

_mesh = plsc.VectorSubcoreMesh(core_axis_name="c", subcore_axis_name="s")


def _nbatches(s):
    return (NBATCH - 1 - s) // 16 + 1


@functools.partial(
    pl.kernel, mesh=_mesh,
    out_type=(jax.ShapeDtypeStruct((NPAD,), jnp.float32),
              jax.ShapeDtypeStruct((NPAD,), jnp.float32)),
    scratch_types=[pltpu.VMEM_SHARED((NPAD,), jnp.float32),
                   pltpu.VMEM((EB,), jnp.float32),
                   pltpu.VMEM((BPT * EB,), jnp.int32),
                   pltpu.VMEM((BPT * EB,), jnp.int32),
                   pltpu.VMEM((EB,), jnp.int32)],
)
def _stats_sc(src_hbm, dst_hbm, z_hbm, ones_hbm, deg_hbm, kill_hbm,
              acc, onesv, sall, dall, midx):
    c = lax.axis_index("c")
    s = lax.axis_index("s")
    pltpu.sync_copy(z_hbm, acc.at[pl.ds(s * RPT, RPT)])
    pltpu.sync_copy(ones_hbm, onesv)
    start = 78 * s + jnp.minimum(s, 2)
    nb = 78 + jnp.where(s < 2, 1, 0)
    pltpu.sync_copy(dst_hbm.at[pl.ds(start * EB, BPT * EB)], dall)

    @pl.when(c == 1)
    def _lds():
        pltpu.sync_copy(src_hbm.at[pl.ds(start * EB, BPT * EB)], sall)

    plsc.subcore_barrier()

    @pl.when(c == 0)
    def _deg():
        def body(i, carry):
            for j in range(EB // 16):
                o = i * EB + 16 * j
                midx[pl.ds(16 * j, 16)] = dall[pl.ds(o, 16)]
            pltpu.sync_copy(onesv, acc.at[midx], add=True)
            return carry
        lax.fori_loop(0, nb, body, 0)

    @pl.when(c == 1)
    def _kill():
        def body(i, carry):
            for j in range(EB // 16):
                o = i * EB + 16 * j
                midx[pl.ds(16 * j, 16)] = jnp.maximum(
                    sall[pl.ds(o, 16)], dall[pl.ds(o, 16)])
            pltpu.sync_copy(onesv, acc.at[midx], add=True)
            return carry
        lax.fori_loop(0, nb, body, 0)

    plsc.subcore_barrier()

    @pl.when(c == 0)
    def _out0():
        pltpu.sync_copy(acc.at[pl.ds(s * RPT, RPT)],
                        deg_hbm.at[pl.ds(s * RPT, RPT)])

    @pl.when(c == 1)
    def _out1():
        pltpu.sync_copy(acc.at[pl.ds(s * RPT, RPT)],
                        kill_hbm.at[pl.ds(s * RPT, RPT)])


@functools.partial(
    pl.kernel, mesh=_mesh,
    out_type=(jax.ShapeDtypeStruct((NPAD, HH), jnp.float32),
              jax.ShapeDtypeStruct((NPAD, HH), jnp.float32)),
    scratch_types=[pltpu.VMEM_SHARED((NPAD, HH), jnp.float32),
                   pltpu.VMEM((EB, HH), jnp.float32),
                   pltpu.VMEM((EB, HH), jnp.float32),
                   pltpu.VMEM((BPT * EB,), jnp.int32),
                   pltpu.VMEM((EB,), jnp.int32),
                   pltpu.VMEM((EB,), jnp.int32),
                   pltpu.SemaphoreType.DMA,
                   pltpu.SemaphoreType.DMA],
)
def _segsum_sc(hp0_hbm, hp1_hbm, src_hbm, dst_hbm, z2_hbm, s0_hbm, s1_hbm,
               acc, rows_a, rows_b, sall, didx_a, didx_b, sem_a, sem_b):
    c = lax.axis_index("c")
    s = lax.axis_index("s")
    pltpu.sync_copy(z2_hbm, acc.at[pl.ds(s * RPT, RPT)])
    start = 78 * s + jnp.minimum(s, 2)
    nb = 78 + jnp.where(s < 2, 1, 0)
    pltpu.sync_copy(src_hbm.at[pl.ds(start * EB, BPT * EB)], sall)
    plsc.subcore_barrier()

    def gather(bi, rows, sem):
        idx = sall.at[pl.ds(bi * EB, EB)]

        @pl.when(c == 0)
        def _g0():
            pltpu.async_copy(hp0_hbm.at[idx], rows, sem)

        @pl.when(c == 1)
        def _g1():
            pltpu.async_copy(hp1_hbm.at[idx], rows, sem)

    def drain(rows, sem):
        pltpu.make_async_copy(hp0_hbm.at[pl.ds(0, EB)], rows, sem).wait()

    def scatter(bi, rows, didx):
        pltpu.sync_copy(dst_hbm.at[pl.ds((start + bi) * EB, EB)], didx)
        pltpu.sync_copy(rows, acc.at[didx], add=True)

    gather(0, rows_a, sem_a)

    def body(i, carry):
        gather(2 * i + 1, rows_b, sem_b)
        drain(rows_a, sem_a)
        scatter(2 * i, rows_a, didx_a)

        @pl.when(2 * i + 2 < nb)
        def _pf():
            gather(2 * i + 2, rows_a, sem_a)

        drain(rows_b, sem_b)
        scatter(2 * i + 1, rows_b, didx_b)
        return carry

    lax.fori_loop(0, 39, body, 0)

    @pl.when(nb > 78)
    def _tail():
        drain(rows_a, sem_a)
        scatter(78, rows_a, didx_a)

    plsc.subcore_barrier()

    @pl.when(c == 0)
    def _out0():
        pltpu.sync_copy(acc.at[pl.ds(s * RPT, RPT)],
                        s0_hbm.at[pl.ds(s * RPT, RPT)])

    @pl.when(c == 1)
    def _out1():
        pltpu.sync_copy(acc.at[pl.ds(s * RPT, RPT)],
                        s1_hbm.at[pl.ds(s * RPT, RPT)])


@functools.partial(
    pl.kernel, mesh=_mesh,
    out_type=jax.ShapeDtypeStruct((CMAX,), jnp.float32),
    scratch_types=[pltpu.VMEM_SHARED((2 * CMAX,), jnp.float32),
                   pltpu.VMEM((RPT,), jnp.float32),
                   pltpu.VMEM((RPT,), jnp.float32),
                   pltpu.VMEM((EB,), jnp.int32),
                   pltpu.VMEM((EB,), jnp.float32)],
)
def _scan_sc(cand_hbm, rank_hbm, z_hbm, clist_hbm, acc2, cb, rb, idx, val):
    c = lax.axis_index("c")
    s = lax.axis_index("s")

    @pl.when(c == 0)
    def _run():
        pltpu.sync_copy(z_hbm.at[pl.ds(0, EB)], acc2.at[pl.ds(s * EB, EB)])
        pltpu.sync_copy(cand_hbm.at[pl.ds(s * RPT, RPT)], cb)
        pltpu.sync_copy(rank_hbm.at[pl.ds(s * RPT, RPT)], rb)
        plsc.subcore_barrier()

        def body(i, carry):
            for j in range(EB // 16):
                o = i * EB + 16 * j
                v = cb[pl.ds(o, 16)]
                rk = rb[pl.ds(o, 16)].astype(jnp.int32)
                nidx = lax.iota(jnp.int32, 16) + (s * RPT + o)
                trash = CMAX + jnp.bitwise_and(nidx, CMAX - 1)
                idx[pl.ds(16 * j, 16)] = jnp.where(v > 0.5, rk, trash)
                val[pl.ds(16 * j, 16)] = nidx.astype(jnp.float32)
            pltpu.sync_copy(val, acc2.at[idx], add=True)
            return carry

        lax.fori_loop(0, RPT // EB, body, 0)
        plsc.subcore_barrier()

        @pl.when(s < CMAX // EB)
        def _out():
            pltpu.sync_copy(acc2.at[pl.ds(s * EB, EB)],
                            clist_hbm.at[pl.ds(s * EB, EB)])


@functools.partial(
    pl.kernel, mesh=_mesh,
    out_type=(jax.ShapeDtypeStruct((CMAX, H), jnp.float32),
              jax.ShapeDtypeStruct((CMAX, 128), jnp.float32)),
    scratch_types=[pltpu.VMEM((EB,), jnp.int32),
                   pltpu.VMEM((EB,), jnp.float32),
                   pltpu.VMEM((EB, H), jnp.float32),
                   pltpu.VMEM((EB, 128), jnp.float32),
                   pltpu.SemaphoreType.DMA],
)
def _gather_sc(clist_hbm, x1_hbm, pos_hbm, x1c_hbm, posc_hbm,
               idxb, clf, rx, rp, sem):
    c = lax.axis_index("c")
    s = lax.axis_index("s")

    @pl.when(s < CMAX // EB)
    def _run():
        pltpu.sync_copy(clist_hbm.at[pl.ds(s * EB, EB)], clf)
        for j in range(EB // 16):
            idxb[pl.ds(16 * j, 16)] = clf[pl.ds(16 * j, 16)].astype(
                jnp.int32)

        @pl.when(c == 0)
        def _x1():
            pltpu.async_copy(x1_hbm.at[idxb], rx, sem).wait()
            pltpu.sync_copy(rx, x1c_hbm.at[pl.ds(s * EB, EB)])

        @pl.when(c == 1)
        def _pos():
            pltpu.async_copy(pos_hbm.at[idxb], rp, sem).wait()
            pltpu.sync_copy(rp, posc_hbm.at[pl.ds(s * EB, EB)])


def _conv_prep_body(x_ref, w_ref, b_ref, deg_ref, kill_ref,
                    hp0_ref, hp1_ref, p1_ref, dinv_ref, cand_ref,
                    rank_ref, mc_ref, carry_ref):
    i = pl.program_id(0)
    h = jnp.dot(x_ref[...], w_ref[...], preferred_element_type=jnp.float32)
    dinv = lax.rsqrt(deg_ref[...] + 2.0)
    hp = dinv * h
    hp0_ref[...] = hp[:, :HH]
    hp1_ref[...] = hp[:, HH:]
    p1_ref[...] = 2.0 * dinv * dinv * h + b_ref[...]
    dinv_ref[...] = dinv

    @pl.when(i == 0)
    def _init():
        carry_ref[0, 0] = 0.0

    ridx = lax.broadcasted_iota(jnp.int32, (RB, 1), 0) + i * RB
    keep = jnp.where(
        jnp.logical_and(kill_ref[...] == 0.0, ridx < N), 1.0, 0.0)
    rr = lax.broadcasted_iota(jnp.int32, (RB, RB), 0)
    cc = lax.broadcasted_iota(jnp.int32, (RB, RB), 1)
    tri = jnp.where(cc <= rr, 1.0, 0.0)
    p = jnp.dot(tri, keep, preferred_element_type=jnp.float32)
    carry = carry_ref[0, 0]
    rank = p - keep + carry
    cand_ref[...] = keep * jnp.where(rank < 1000.0, 1.0, 0.0)
    rank_ref[...] = rank
    new_carry = carry + jnp.sum(keep)
    carry_ref[0, 0] = new_carry

    @pl.when(i == GRID - 1)
    def _emit_m():
        mc_ref[...] = jnp.zeros((1, 16), jnp.float32) + jnp.minimum(
            new_carry, 1000.0)


def _conv_prep(x, w, b2, deg2, kill2):
    return pl.pallas_call(
        _conv_prep_body,
        grid=(GRID,),
        in_specs=[pl.BlockSpec((RB, H), lambda i: (i, 0)),
                  pl.BlockSpec((H, H), lambda i: (0, 0)),
                  pl.BlockSpec((1, H), lambda i: (0, 0)),
                  pl.BlockSpec((RB, 1), lambda i: (i, 0)),
                  pl.BlockSpec((RB, 1), lambda i: (i, 0))],
        out_specs=[pl.BlockSpec((RB, HH), lambda i: (i, 0)),
                   pl.BlockSpec((RB, HH), lambda i: (i, 0)),
                   pl.BlockSpec((RB, H), lambda i: (i, 0)),
                   pl.BlockSpec((RB, 1), lambda i: (i, 0)),
                   pl.BlockSpec((RB, 1), lambda i: (i, 0)),
                   pl.BlockSpec((RB, 1), lambda i: (i, 0)),
                   pl.BlockSpec((1, 16), lambda i: (0, 0))],
        out_shape=[jax.ShapeDtypeStruct((NPAD, HH), jnp.float32),
                   jax.ShapeDtypeStruct((NPAD, HH), jnp.float32),
                   jax.ShapeDtypeStruct((NPAD, H), jnp.float32),
                   jax.ShapeDtypeStruct((NPAD, 1), jnp.float32),
                   jax.ShapeDtypeStruct((NPAD, 1), jnp.float32),
                   jax.ShapeDtypeStruct((NPAD, 1), jnp.float32),
                   jax.ShapeDtypeStruct((1, 16), jnp.float32)],
        scratch_shapes=[pltpu.SMEM((1, 1), jnp.float32)],
    )(x, w, b2, deg2, kill2)


def _finish_body(s0_ref, s1_ref, p_ref, dinv_ref, o_ref):
    sall = jnp.concatenate([s0_ref[...], s1_ref[...]], axis=1)
    o_ref[...] = jnp.maximum(dinv_ref[...] * sall + p_ref[...], 0.0)


def _conv_finish(s0, s1, p, dinv2):
    return pl.pallas_call(
        _finish_body,
        grid=(GRID,),
        in_specs=[pl.BlockSpec((RB, HH), lambda i: (i, 0)),
                  pl.BlockSpec((RB, HH), lambda i: (i, 0)),
                  pl.BlockSpec((RB, H), lambda i: (i, 0)),
                  pl.BlockSpec((RB, 1), lambda i: (i, 0))],
        out_specs=pl.BlockSpec((RB, H), lambda i: (i, 0)),
        out_shape=jax.ShapeDtypeStruct((NPAD, H), jnp.float32),
    )(s0, s1, p, dinv2)


def _interp_body(posq_ref, posc_ref, m_ref, x1c_ref, wup_ref, bup_ref,
                 dinv_ref, hp0_ref, hp1_ref, p2_ref):
    pq = posq_ref[...]
    pc = posc_ref[...]
    dmat = jnp.zeros((RB, CMAX), jnp.float32)
    for k in range(3):
        diff = pq[:, k:k + 1] - pc[:, k][None, :]
        dmat = dmat + diff * diff
    m = m_ref[0, 0]
    col = lax.broadcasted_iota(jnp.int32, (RB, CMAX), 1).astype(jnp.float32)
    dmat = jnp.where(col < m, dmat, jnp.inf)
    wmat = jnp.zeros((RB, CMAX), jnp.float32)
    den = jnp.zeros((RB, 1), jnp.float32)
    for _ in range(KNN):
        mn = jnp.min(dmat, axis=1, keepdims=True)
        w = 1.0 / jnp.maximum(mn, 1e-16)
        cstar = jnp.min(jnp.where(dmat <= mn, col, 2.0 * CMAX),
                        axis=1, keepdims=True)
        sel = col == cstar
        wmat = wmat + jnp.where(sel, w, 0.0)
        den = den + w
        dmat = jnp.where(sel, jnp.inf, dmat)
    xu = jnp.dot(wmat, x1c_ref[...],
                 preferred_element_type=jnp.float32) / den
    h2 = jnp.dot(xu, wup_ref[...], preferred_element_type=jnp.float32)
    dinv = dinv_ref[...]
    hp = dinv * h2
    hp0_ref[...] = hp[:, :HH]
    hp1_ref[...] = hp[:, HH:]
    p2_ref[...] = 2.0 * dinv * dinv * h2 + bup_ref[...]


def _interp(pos16, posc, m16, x1c, wup, bup2, dinv2):
    return pl.pallas_call(
        _interp_body,
        grid=(GRID,),
        in_specs=[pl.BlockSpec((RB, 16), lambda i: (i, 0)),
                  pl.BlockSpec((CMAX, 128), lambda i: (0, 0)),
                  pl.BlockSpec((1, 16), lambda i: (0, 0)),
                  pl.BlockSpec((CMAX, H), lambda i: (0, 0)),
                  pl.BlockSpec((H, H), lambda i: (0, 0)),
                  pl.BlockSpec((1, H), lambda i: (0, 0)),
                  pl.BlockSpec((RB, 1), lambda i: (i, 0))],
        out_specs=[pl.BlockSpec((RB, HH), lambda i: (i, 0)),
                   pl.BlockSpec((RB, HH), lambda i: (i, 0)),
                   pl.BlockSpec((RB, H), lambda i: (i, 0))],
        out_shape=[jax.ShapeDtypeStruct((NPAD, HH), jnp.float32),
                   jax.ShapeDtypeStruct((NPAD, HH), jnp.float32),
                   jax.ShapeDtypeStruct((NPAD, H), jnp.float32)],
    )(pos16, posc, m16, x1c, wup, bup2, dinv2)


def _final_body(s0_ref, s1_ref, p_ref, dinv_ref, wl_ref, bl_ref, o_ref):
    sall = jnp.concatenate([s0_ref[...], s1_ref[...]], axis=1)
    xu2 = jnp.maximum(dinv_ref[...] * sall + p_ref[...], 0.0)
    o_ref[...] = jnp.dot(xu2, wl_ref[...],
                         preferred_element_type=jnp.float32) + bl_ref[...]


def _final(s0, s1, p, dinv2, wl, bl2):
    return pl.pallas_call(
        _final_body,
        grid=(GRID,),
        in_specs=[pl.BlockSpec((RB, HH), lambda i: (i, 0)),
                  pl.BlockSpec((RB, HH), lambda i: (i, 0)),
                  pl.BlockSpec((RB, H), lambda i: (i, 0)),
                  pl.BlockSpec((RB, 1), lambda i: (i, 0)),
                  pl.BlockSpec((H, H), lambda i: (0, 0)),
                  pl.BlockSpec((1, H), lambda i: (0, 0))],
        out_specs=pl.BlockSpec((RB, H), lambda i: (i, 0)),
        out_shape=jax.ShapeDtypeStruct((NPAD, H), jnp.float32),
    )(s0, s1, p, dinv2, wl, bl2)


@jax.jit
def kernel(x, pos, edge_index, batch, W_down, b_down, W_up, b_up,
           W_lin, b_lin):
    src = jnp.pad(edge_index[0].astype(jnp.int32), (0, EPAD - E))
    dst = jnp.pad(edge_index[1].astype(jnp.int32), (0, EPAD - E))
    x_pad = jnp.pad(x, ((0, NPAD - N), (0, 0)))
    pos16 = jnp.pad(pos, ((0, NPAD - N), (0, 13)))
    pos128 = jnp.pad(pos, ((0, NPAD - N), (0, 125)))
    z1 = jnp.zeros((RPT,), jnp.float32)
    z2 = jnp.zeros((RPT, HH), jnp.float32)
    ones = jnp.ones((EB,), jnp.float32)

    deg, kill = _stats_sc(src, dst, z1, ones)
    hp0, hp1, p1, dinv2, cand2, rank2, mcount = _conv_prep(
        x_pad, W_down, b_down[None, :], deg[:, None], kill[:, None])
    s0, s1 = _segsum_sc(hp0, hp1, src, dst, z2)
    clist = _scan_sc(cand2[:, 0], rank2[:, 0], z1)
    x1 = _conv_finish(s0, s1, p1, dinv2)
    x1c, posc = _gather_sc(clist, x1, pos128)
    h2p0, h2p1, p2 = _interp(pos16, posc, mcount, x1c,
                             W_up, b_up[None, :], dinv2)
    s20, s21 = _segsum_sc(h2p0, h2p1, src, dst, z2)
    out = _final(s20, s21, p2, dinv2, W_lin, b_lin[None, :])
    return out[:N]
